# baseline (speedup 1.0000x reference)
import functools

import jax
import jax.numpy as jnp
from jax import lax
from jax.experimental import pallas as pl
from jax.experimental.pallas import tpu as pltpu
from jax.experimental.pallas import tpu_sc as plsc

NUM_CORES = 2
NUM_SUBCORES = 16
NUM_WORKERS = NUM_CORES * NUM_SUBCORES
CHUNK = 512
NBUF = 3


@functools.partial(jax.jit, static_argnames=("n_chunks",))
def _sc_embedding_gather(idx3, table, n_chunks):
    D = table.shape[1]
    mesh = plsc.VectorSubcoreMesh(core_axis_name="c", subcore_axis_name="s")

    @functools.partial(
        pl.kernel,
        mesh=mesh,
        out_type=jax.ShapeDtypeStruct(
            (NUM_WORKERS, n_chunks, CHUNK, D), jnp.float32),
        scratch_types=[
            pltpu.VMEM((n_chunks, CHUNK), jnp.int32),
            pltpu.VMEM((NBUF, CHUNK, D), jnp.float32),
            pltpu.SemaphoreType.DMA((NBUF,)),
            pltpu.SemaphoreType.DMA((NBUF,)),
            pltpu.SemaphoreType.DMA,
        ],
        compiler_params=pltpu.CompilerParams(use_tc_tiling_on_sc=False),
    )
    def k(idx_hbm, table_hbm, out_hbm, idx_v, bufs, gsem, osem, isem):
        wid = lax.axis_index("s") * NUM_CORES + lax.axis_index("c")
        pltpu.async_copy(idx_hbm.at[wid], idx_v, isem).wait()

        def gather(j, b):
            return pltpu.make_async_copy(table_hbm.at[idx_v.at[j]],
                                         bufs.at[b], gsem.at[b])

        def out_copy(j, b):
            return pltpu.make_async_copy(bufs.at[b], out_hbm.at[wid, j],
                                         osem.at[b])

        for b in range(NBUF):
            gather(b, b).start()

        def body(j, carry):
            b = lax.rem(j, NBUF)
            gather(j, b).wait()
            out_copy(j, b).start()
            nxt = j + NBUF

            @pl.when(nxt < n_chunks)
            def _():
                out_copy(j, b).wait()
                gather(nxt, b).start()

            return carry

        lax.fori_loop(0, n_chunks, body, 0)

        for i in range(NBUF):
            j = n_chunks - NBUF + i
            out_copy(j, j % NBUF).wait()

    return k(idx3, table)


def kernel(token_ids, embedding):
    orig_shape = token_ids.shape
    flat = token_ids.reshape(-1).astype(jnp.int32)
    n = flat.shape[0]
    group = NUM_WORKERS * CHUNK
    n_pad = -n % (group * NBUF)
    if n_pad:
        flat = jnp.concatenate([flat, jnp.zeros((n_pad,), jnp.int32)])
    n_chunks = (n + n_pad) // group
    idx3 = flat.reshape(NUM_WORKERS, n_chunks, CHUNK)
    rows = _sc_embedding_gather(idx3, embedding, n_chunks)
    rows = rows.reshape(-1, embedding.shape[1])
    if n_pad:
        rows = rows[:n]
    return rows.reshape(*orig_shape, embedding.shape[1])


# speedup vs baseline: 1.9232x; 1.9232x over previous
import functools

import jax
import jax.numpy as jnp
from jax import lax
from jax.experimental import pallas as pl
from jax.experimental.pallas import tpu as pltpu
from jax.experimental.pallas import tpu_sc as plsc

NUM_CORES = 2
NUM_SUBCORES = 16
NUM_WORKERS = NUM_CORES * NUM_SUBCORES
CHUNK = 128
NBUF = 8
GAHEAD = 4


@functools.partial(jax.jit, static_argnames=("n_chunks",))
def _sc_embedding_gather(idx3, table, n_chunks):
    D = table.shape[1]
    mesh = plsc.VectorSubcoreMesh(core_axis_name="c", subcore_axis_name="s")

    @functools.partial(
        pl.kernel,
        mesh=mesh,
        out_type=jax.ShapeDtypeStruct(
            (NUM_WORKERS, n_chunks, CHUNK, D), jnp.float32),
        scratch_types=[
            pltpu.VMEM((n_chunks, CHUNK), jnp.int32),
            pltpu.VMEM((NBUF, CHUNK, D), jnp.float32),
            pltpu.SemaphoreType.DMA((NBUF,)),
            pltpu.SemaphoreType.DMA((NBUF,)),
            pltpu.SemaphoreType.DMA,
        ],
        compiler_params=pltpu.CompilerParams(use_tc_tiling_on_sc=False),
    )
    def k(idx_hbm, table_hbm, out_hbm, idx_v, bufs, gsem, osem, isem):
        wid = lax.axis_index("s") * NUM_CORES + lax.axis_index("c")
        pltpu.async_copy(idx_hbm.at[wid], idx_v, isem).wait()

        def gather(j, b):
            return pltpu.make_async_copy(table_hbm.at[idx_v.at[j]],
                                         bufs.at[b], gsem.at[b])

        def out_copy(j, b):
            return pltpu.make_async_copy(bufs.at[b], out_hbm.at[wid, j],
                                         osem.at[b])

        for b in range(GAHEAD):
            gather(b, b).start()

        def body(j, carry):
            b = lax.rem(j, NBUF)
            gather(j, b).wait()
            out_copy(j, b).start()
            nxt = j + GAHEAD

            @pl.when(nxt < n_chunks)
            def _():
                nb = lax.rem(nxt, NBUF)

                @pl.when(nxt >= NBUF)
                def _():
                    out_copy(nxt - NBUF, nb).wait()

                gather(nxt, nb).start()

            return carry

        lax.fori_loop(0, n_chunks, body, 0)

        for i in range(NBUF):
            j = n_chunks - NBUF + i
            out_copy(j, j % NBUF).wait()

    return k(idx3, table)


def kernel(token_ids, embedding):
    orig_shape = token_ids.shape
    flat = token_ids.reshape(-1).astype(jnp.int32)
    n = flat.shape[0]
    group = NUM_WORKERS * CHUNK
    n_chunks = max(-(-n // group), NBUF)
    n_pad = n_chunks * group - n
    if n_pad:
        flat = jnp.concatenate([flat, jnp.zeros((n_pad,), jnp.int32)])
    idx3 = flat.reshape(NUM_WORKERS, n_chunks, CHUNK)
    rows = _sc_embedding_gather(idx3, embedding, n_chunks)
    rows = rows.reshape(-1, embedding.shape[1])
    if n_pad:
        rows = rows[:n]
    return rows.reshape(*orig_shape, embedding.shape[1])
